# baseline (speedup 1.0000x reference)
import functools

import jax
import jax.numpy as jnp
from jax import lax
from jax.experimental import pallas as pl
from jax.experimental.pallas import tpu as pltpu
from jax.experimental.pallas import tpu_sc as plsc

N = 10000
E = 160000
D = 256
H = 512
G = 64
C = 10

NTILE = 32
L = 16
NH = N // 2
CH = 640
BLK = 1000
NGRID = N // BLK
EPS = 1e-5


def _sc_edge_max(btab, src, dst):
    mesh = plsc.VectorSubcoreMesh(core_axis_name="c", subcore_axis_name="s")

    @functools.partial(
        pl.kernel,
        mesh=mesh,
        compiler_params=pltpu.CompilerParams(use_tc_tiling_on_sc=False),
        out_type=jax.ShapeDtypeStruct((NTILE, N, L), jnp.float32),
        scratch_types=[
            pltpu.VMEM((CH,), jnp.int32),
            pltpu.VMEM((CH,), jnp.int32),
            pltpu.VMEM((CH,), jnp.int32),
            pltpu.VMEM((CH, L), jnp.float32),
            pltpu.VMEM((NH, L), jnp.float32),
            pltpu.SemaphoreType.DMA,
        ],
    )
    def k(btab_hbm, src_hbm, dst_hbm, mt_hbm, srcs_v, dsts_v, idx_v, rows_v,
          acc_v, sem):
        wid = lax.axis_index("s") * 2 + lax.axis_index("c")

        for half in range(2):
            base = half * NH

            def init_body(i, _):
                acc_v[i] = jnp.full((L,), -jnp.inf, jnp.float32)
                return 0

            lax.fori_loop(0, NH, init_body, 0)

            def chunk_body(c, _):
                off = c * CH
                pltpu.sync_copy(src_hbm.at[pl.ds(off, CH)], srcs_v)
                pltpu.sync_copy(dst_hbm.at[pl.ds(off, CH)], dsts_v)

                def idx_body(j, _):
                    s16 = srcs_v[pl.ds(j * L, L)]
                    idx_v[pl.ds(j * L, L)] = s16 * NTILE + wid
                    return 0

                lax.fori_loop(0, CH // L, idx_body, 0)
                pltpu.async_copy(btab_hbm.at[idx_v], rows_v, sem).wait()

                def edge_grp(j, _):
                    dv = dsts_v[pl.ds(j * L, L)]
                    for i in range(L):
                        d = dv[i]

                        @pl.when((d >= base) & (d < base + NH))
                        def _():
                            r = acc_v[d - base]
                            acc_v[d - base] = jnp.maximum(r, rows_v[j * L + i])

                    return 0

                lax.fori_loop(0, CH // L, edge_grp, 0)
                return 0

            lax.fori_loop(0, E // CH, chunk_body, 0)
            pltpu.sync_copy(acc_v, mt_hbm.at[wid, pl.ds(base, NH)])

    return k(btab, src, dst)


def _mm_first_body(x_ref, wd_ref, wb_ref, bias_ref, a_ref, b_ref):
    z = x_ref[...]
    a_ref[...] = (
        jnp.dot(z, wd_ref[...], preferred_element_type=jnp.float32)
        + bias_ref[...]
    )
    b_ref[...] = jnp.dot(z, wb_ref[...], preferred_element_type=jnp.float32)


def _mm_bn_body(x_ref, st_ref, g_ref, bt_ref, wd_ref, wb_ref, bias_ref,
                a_ref, b_ref):
    m = st_ref[0:1, :] / N
    v = st_ref[1:2, :] / N - m * m
    sc = g_ref[...] * lax.rsqrt(v + EPS)
    sh = bt_ref[...] - m * sc
    z = jnp.maximum(x_ref[...] * sc + sh, 0.0)
    a_ref[...] = (
        jnp.dot(z, wd_ref[...], preferred_element_type=jnp.float32)
        + bias_ref[...]
    )
    b_ref[...] = jnp.dot(z, wb_ref[...], preferred_element_type=jnp.float32)


def _epi_body(a_ref, m_ref, h_ref, st_ref):
    h = jnp.maximum(a_ref[...] + m_ref[...], 0.0)
    h_ref[...] = h

    @pl.when(pl.program_id(0) == 0)
    def _():
        st_ref[...] = jnp.zeros_like(st_ref)

    st_ref[0:1, :] = st_ref[0:1, :] + jnp.sum(h, axis=0, keepdims=True)
    st_ref[1:2, :] = st_ref[1:2, :] + jnp.sum(h * h, axis=0, keepdims=True)


def _head_body(h_ref, st_ref, g_ref, bt_ref, batch_ref, w4_ref, b4_ref,
               out_ref, yacc, cacc):
    i = pl.program_id(0)

    @pl.when(i == 0)
    def _():
        yacc[...] = jnp.zeros_like(yacc)
        cacc[...] = jnp.zeros_like(cacc)

    m = st_ref[0:1, :] / N
    v = st_ref[1:2, :] / N - m * m
    sc = g_ref[...] * lax.rsqrt(v + EPS)
    sh = bt_ref[...] - m * sc
    z = h_ref[...] * sc + sh
    y = jnp.dot(z, w4_ref[...], preferred_element_type=jnp.float32)

    b = jnp.reshape(batch_ref[...], (1, BLK))
    p = (lax.broadcasted_iota(jnp.int32, (G, BLK), 0) == b).astype(jnp.float32)
    yacc[...] = yacc[...] + jnp.dot(p, y, preferred_element_type=jnp.float32)
    cacc[...] = cacc[...] + jnp.sum(p, axis=1, keepdims=True)

    @pl.when(i == NGRID - 1)
    def _():
        pooled = yacc[...] / jnp.maximum(cacc[...], 1.0)
        logits = pooled + b4_ref[...]
        col = lax.broadcasted_iota(jnp.int32, (G, 128), 1)
        logits = jnp.where(col < C, logits, -1e30)
        mx = jnp.max(logits, axis=1, keepdims=True)
        e = jnp.exp(logits - mx)
        sm = e / jnp.sum(e, axis=1, keepdims=True)
        out_ref[...] = sm[:, :C]


def _row_spec(width):
    return pl.BlockSpec((BLK, width), lambda i: (i, 0))


def _whole(shape):
    nd = len(shape)
    return pl.BlockSpec(shape, lambda i: (0,) * nd)


def _mm_first(x, wd, wb, bias):
    k = x.shape[1]
    return pl.pallas_call(
        _mm_first_body,
        grid=(NGRID,),
        in_specs=[_row_spec(k), _whole((k, H)), _whole((k, H)),
                  _whole((1, H))],
        out_specs=[_row_spec(H), _row_spec(H)],
        out_shape=[jax.ShapeDtypeStruct((N, H), jnp.float32)] * 2,
    )(x, wd, wb, bias)


def _mm_bn(h, st, g, bt, wd, wb, bias):
    return pl.pallas_call(
        _mm_bn_body,
        grid=(NGRID,),
        in_specs=[_row_spec(H), _whole((8, H)), _whole((1, H)),
                  _whole((1, H)), _whole((H, H)), _whole((H, H)),
                  _whole((1, H))],
        out_specs=[_row_spec(H), _row_spec(H)],
        out_shape=[jax.ShapeDtypeStruct((N, H), jnp.float32)] * 2,
    )(h, st, g, bt, wd, wb, bias)


def _epilogue(a, m):
    return pl.pallas_call(
        _epi_body,
        grid=(NGRID,),
        in_specs=[_row_spec(H), _row_spec(H)],
        out_specs=[_row_spec(H), _whole((8, H))],
        out_shape=[jax.ShapeDtypeStruct((N, H), jnp.float32),
                   jax.ShapeDtypeStruct((8, H), jnp.float32)],
    )(a, m)


def _head(h, st, g, bt, batch3, w4p, b4p):
    return pl.pallas_call(
        _head_body,
        grid=(NGRID,),
        in_specs=[_row_spec(H), _whole((8, H)), _whole((1, H)),
                  _whole((1, H)),
                  pl.BlockSpec((1, 1, BLK), lambda i: (i, 0, 0)),
                  _whole((H, 128)), _whole((1, 128))],
        out_specs=pl.BlockSpec((G, C), lambda i: (0, 0)),
        out_shape=jax.ShapeDtypeStruct((G, C), jnp.float32),
        scratch_shapes=[pltpu.VMEM((G, 128), jnp.float32),
                        pltpu.VMEM((G, 128), jnp.float32)],
    )(h, st, g, bt, batch3, w4p, b4p)


def _layer_inputs(w, b, din):
    wd = w[:din] - w[din:]
    wb = w[din:]
    return wd, wb, b.reshape(1, H)


def _untranspose(mt):
    return jnp.transpose(mt, (1, 0, 2)).reshape(N, H)


def kernel(x, edge_index, batch, W1, b1, g1, bt1, W2, b2, g2, bt2,
           W3, b3, g3, bt3, W4, b4):
    src = edge_index[0]
    dst = edge_index[1]

    wd1, wb1, bb1 = _layer_inputs(W1, b1, D)
    wd2, wb2, bb2 = _layer_inputs(W2, b2, H)
    wd3, wb3, bb3 = _layer_inputs(W3, b3, H)
    w4p = jnp.zeros((H, 128), jnp.float32).at[:, :C].set(W4)
    b4p = jnp.zeros((1, 128), jnp.float32).at[:, :C].set(b4)
    batch3 = batch.reshape(NGRID, 1, BLK)

    a1, bmat1 = _mm_first(x, wd1, wb1, bb1)
    m1 = _untranspose(_sc_edge_max(bmat1.reshape(N * NTILE, L), src, dst))
    h1, st1 = _epilogue(a1, m1)

    a2, bmat2 = _mm_bn(h1, st1, g1.reshape(1, H), bt1.reshape(1, H),
                       wd2, wb2, bb2)
    m2 = _untranspose(_sc_edge_max(bmat2.reshape(N * NTILE, L), src, dst))
    h2, st2 = _epilogue(a2, m2)

    a3, bmat3 = _mm_bn(h2, st2, g2.reshape(1, H), bt2.reshape(1, H),
                       wd3, wb3, bb3)
    m3 = _untranspose(_sc_edge_max(bmat3.reshape(N * NTILE, L), src, dst))
    h3, st3 = _epilogue(a3, m3)

    return _head(h3, st3, g3.reshape(1, H), bt3.reshape(1, H),
                 batch3, w4p, b4p)


# speedup vs baseline: 2.0671x; 2.0671x over previous
import functools

import jax
import jax.numpy as jnp
from jax import lax
from jax.experimental import pallas as pl
from jax.experimental.pallas import tpu as pltpu
from jax.experimental.pallas import tpu_sc as plsc

N = 10000
E = 160000
D = 256
H = 512
G = 64
C = 10

NTILE = 32
L = 16
NH = N // 2
CH = 512
TE = E // NTILE
TEP = TE + 16
CAP = 5024
NREG = NTILE * 2
TABN = 544
TRASH = 528
BLK = 1000
NGRID = N // BLK
EPS = 1e-5
_SC_PARAMS = pltpu.CompilerParams(use_tc_tiling_on_sc=False,
                                  needs_layout_passes=False)


def _mesh():
    return plsc.VectorSubcoreMesh(core_axis_name="c", subcore_axis_name="s")


def _wid():
    return lax.axis_index("s") * 2 + lax.axis_index("c")


def _sc_bucket(src, dst):
    @functools.partial(
        pl.kernel,
        mesh=_mesh(),
        compiler_params=_SC_PARAMS,
        out_type=[
            jax.ShapeDtypeStruct((NTILE, 2, CAP), jnp.int32),
            jax.ShapeDtypeStruct((NTILE, 2, CAP), jnp.int32),
            jax.ShapeDtypeStruct((NTILE, L), jnp.int32),
        ],
        scratch_types=[
            pltpu.VMEM((TEP,), jnp.int32),
            pltpu.VMEM((TEP,), jnp.int32),
            pltpu.VMEM((CAP + L,), jnp.int32),
            pltpu.VMEM((CAP + L,), jnp.int32),
            pltpu.VMEM((CAP + L,), jnp.int32),
            pltpu.VMEM((CAP + L,), jnp.int32),
            pltpu.VMEM((L,), jnp.int32),
        ],
    )
    def k(src_hbm, dst_hbm, bsrc_hbm, bdst_hbm, cnt_hbm,
          sin, din, s0, d0, s1, d1, stage):
        wid = _wid()
        pltpu.sync_copy(src_hbm.at[pl.ds(wid * TE, TE)], sin.at[pl.ds(0, TE)])
        pltpu.sync_copy(dst_hbm.at[pl.ds(wid * TE, TE)], din.at[pl.ds(0, TE)])

        zeros = jnp.zeros((L,), jnp.int32)
        dumdst = jnp.full((L,), N, jnp.int32)

        def prefill(i, _):
            s0[pl.ds(i * L, L)] = zeros
            s1[pl.ds(i * L, L)] = zeros
            d0[pl.ds(i * L, L)] = dumdst
            d1[pl.ds(i * L, L)] = dumdst
            return 0

        lax.fori_loop(0, (CAP + L) // L, prefill, 0)

        sin[pl.ds(TE, L)] = zeros
        din[pl.ds(TE, L)] = dumdst

        iota = lax.iota(jnp.int32, L)
        ngrp = (TE + L - 1) // L

        def compact(j, carry):
            c0, c1 = carry
            sv = sin[pl.ds(j * L, L)]
            dv = din[pl.ds(j * L, L)]
            m1 = dv >= NH
            k1 = m1.astype(jnp.int32)
            k0 = 1 - k1
            p0 = plsc.cumsum(k0)
            p1 = plsc.cumsum(k1)
            idx0 = jnp.where(m1, CAP, c0 + p0 - 1)
            idx1 = jnp.where(m1, c1 + p1 - 1, CAP)
            plsc.store_scatter(s0, [idx0], sv)
            plsc.store_scatter(d0, [idx0], dv)
            plsc.store_scatter(s1, [idx1], sv)
            plsc.store_scatter(d1, [idx1], dv)
            n1 = jnp.sum(k1)
            return c0 + (L - n1), c1 + n1

        c0, c1 = lax.fori_loop(0, ngrp, compact, (jnp.int32(0), jnp.int32(0)))

        pltpu.sync_copy(s0.at[pl.ds(0, CAP)], bsrc_hbm.at[wid, 0])
        pltpu.sync_copy(d0.at[pl.ds(0, CAP)], bdst_hbm.at[wid, 0])
        pltpu.sync_copy(s1.at[pl.ds(0, CAP)], bsrc_hbm.at[wid, 1])
        pltpu.sync_copy(d1.at[pl.ds(0, CAP)], bdst_hbm.at[wid, 1])
        stage[...] = (jnp.where(iota == 0, c0, 0)
                      + jnp.where(iota == 1, c1, 0))
        pltpu.sync_copy(stage, cnt_hbm.at[wid])

    return k(src, dst)


def _sc_edge_max(btab, bsrc, bdst, counts):
    @functools.partial(
        pl.kernel,
        mesh=_mesh(),
        compiler_params=_SC_PARAMS,
        out_type=jax.ShapeDtypeStruct((NTILE, N, L), jnp.float32),
        scratch_types=[
            pltpu.VMEM((NTILE, L), jnp.int32),
            pltpu.VMEM((TABN,), jnp.int32),
            pltpu.VMEM((TABN,), jnp.int32),
            pltpu.VMEM((CH,), jnp.int32),
            pltpu.VMEM((CH,), jnp.int32),
            pltpu.VMEM((CH,), jnp.int32),
            pltpu.VMEM((CH,), jnp.int32),
            pltpu.VMEM((CH,), jnp.int32),
            pltpu.VMEM((CH,), jnp.int32),
            pltpu.VMEM((CH, L), jnp.float32),
            pltpu.VMEM((CH, L), jnp.float32),
            pltpu.VMEM((NH, L), jnp.float32),
            pltpu.SemaphoreType.DMA,
            pltpu.SemaphoreType.DMA,
            pltpu.SemaphoreType.DMA,
            pltpu.SemaphoreType.DMA,
            pltpu.SemaphoreType.DMA,
            pltpu.SemaphoreType.DMA,
        ],
    )
    def k(btab_hbm, bsrc_hbm, bdst_hbm, cnt_hbm, mt_hbm,
          cnt_v, tab0, tab1, srcA, srcB, dstA, dstB, idxA, idxB,
          rowA, rowB, acc_v,
          sA, sB, dA, dB, gA, gB):
        wid = _wid()
        iota = lax.iota(jnp.int32, L)
        pltpu.sync_copy(cnt_hbm, cnt_v)

        srcs = (srcA, srcB)
        dsts = (dstA, dstB)
        idxs = (idxA, idxB)
        rows = (rowA, rowB)
        ssems = (sA, sB)
        dsems = (dA, dB)
        gsems = (gA, gB)
        tabs = (tab0, tab1)

        def build(p):
            def region(t2, nc):
                cnt = cnt_v[t2][p]
                ncr = (cnt + (CH - 1)) // CH
                base = (t2 * 2 + p) * CAP
                tidx = jnp.where(iota < ncr, nc + iota, TRASH)
                plsc.store_scatter(tabs[p], [tidx], base + iota * CH)
                return nc + ncr

            return lax.fori_loop(0, NTILE, region, jnp.int32(0))

        ncs = (build(0), build(1))

        def tab_at(p, c):
            cb = (c // 8) * 8
            v = tabs[p][pl.ds(cb, L)]
            raw = jnp.sum(jnp.where(iota == c - cb, v, 0))
            return (raw // 16) * 16

        def start_ids(p, c, b):
            off = tab_at(p, c)
            cs = pltpu.async_copy(
                bsrc_hbm.at[pl.ds(off, CH)], srcs[b], ssems[b])
            cd = pltpu.async_copy(
                bdst_hbm.at[pl.ds(off, CH)], dsts[b], dsems[b])
            return cs, cd

        def wait_ids(b):
            pltpu.make_async_copy(
                bsrc_hbm.at[pl.ds(0, CH)], srcs[b], ssems[b]).wait()
            pltpu.make_async_copy(
                bdst_hbm.at[pl.ds(0, CH)], dsts[b], dsems[b]).wait()

        def start_gather(b):
            def idx_body(j, _):
                s16 = srcs[b][pl.ds(j * L, L)]
                idxs[b][pl.ds(j * L, L)] = s16 * NTILE + wid
                return 0

            lax.fori_loop(0, CH // L, idx_body, 0)
            pltpu.async_copy(btab_hbm.at[idxs[b]], rows[b], gsems[b])

        def wait_gather(b):
            pltpu.make_async_copy(
                btab_hbm.at[idxs[b]], rows[b], gsems[b]).wait()

        for p in range(2):
            base = p * NH
            nc = ncs[p]

            neg = jnp.full((L,), -jnp.inf, jnp.float32)

            def init_body(i, _):
                acc_v[i] = neg
                return 0

            lax.fori_loop(0, NH, init_body, 0)

            @pl.when(nc > 0)
            def _():
                start_ids(p, 0, 0)
                wait_ids(0)
                start_gather(0)

            @pl.when(nc > 1)
            def _():
                start_ids(p, 1, 1)

            def rmw(c, b):
                def grp(j, _):
                    dv = dsts[b][pl.ds(j * L, L)]
                    for i in range(L):
                        draw = dv[i]
                        dc = jnp.clip(draw - base, 0, NH - 1)
                        va = (draw >= base) & (draw < base + NH)
                        r = rows[b][j * L + i]
                        a = acc_v[dc]
                        acc_v[dc] = jnp.where(va, jnp.maximum(a, r), a)
                    return 0

                lax.fori_loop(0, CH // L, grp, 0)

            def body(c, cur, nxt):
                @pl.when(c < nc)
                def _():
                    @pl.when(c + 1 < nc)
                    def _():
                        wait_ids(nxt)
                        start_gather(nxt)

                    wait_gather(cur)
                    rmw(c, cur)

                    @pl.when(c + 2 < nc)
                    def _():
                        start_ids(p, c + 2, cur)

            def pair(kk, _):
                body(2 * kk, 0, 1)
                body(2 * kk + 1, 1, 0)
                return 0

            lax.fori_loop(0, (nc + 1) // 2, pair, 0)
            pltpu.sync_copy(acc_v, mt_hbm.at[wid, pl.ds(base, NH)])

    return k(btab, bsrc.reshape(-1), bdst.reshape(-1), counts)


def _mm_first_body(x_ref, wd_ref, wb_ref, bias_ref, a_ref, b_ref):
    z = x_ref[...]
    a_ref[...] = (
        jnp.dot(z, wd_ref[...], preferred_element_type=jnp.float32)
        + bias_ref[...]
    )
    b_ref[...] = jnp.dot(z, wb_ref[...], preferred_element_type=jnp.float32)


def _mm_bn_body(x_ref, st_ref, g_ref, bt_ref, wd_ref, wb_ref, bias_ref,
                a_ref, b_ref):
    m = st_ref[0:1, :] / N
    v = st_ref[1:2, :] / N - m * m
    sc = g_ref[...] * lax.rsqrt(v + EPS)
    sh = bt_ref[...] - m * sc
    z = jnp.maximum(x_ref[...] * sc + sh, 0.0)
    a_ref[...] = (
        jnp.dot(z, wd_ref[...], preferred_element_type=jnp.float32)
        + bias_ref[...]
    )
    b_ref[...] = jnp.dot(z, wb_ref[...], preferred_element_type=jnp.float32)


def _epi_body(a_ref, m_ref, h_ref, st_ref):
    h = jnp.maximum(a_ref[...] + m_ref[...], 0.0)
    h_ref[...] = h

    @pl.when(pl.program_id(0) == 0)
    def _():
        st_ref[...] = jnp.zeros_like(st_ref)

    st_ref[0:1, :] = st_ref[0:1, :] + jnp.sum(h, axis=0, keepdims=True)
    st_ref[1:2, :] = st_ref[1:2, :] + jnp.sum(h * h, axis=0, keepdims=True)


def _head_body(h_ref, st_ref, g_ref, bt_ref, batch_ref, w4_ref, b4_ref,
               out_ref, yacc, cacc):
    i = pl.program_id(0)

    @pl.when(i == 0)
    def _():
        yacc[...] = jnp.zeros_like(yacc)
        cacc[...] = jnp.zeros_like(cacc)

    m = st_ref[0:1, :] / N
    v = st_ref[1:2, :] / N - m * m
    sc = g_ref[...] * lax.rsqrt(v + EPS)
    sh = bt_ref[...] - m * sc
    z = h_ref[...] * sc + sh
    y = jnp.dot(z, w4_ref[...], preferred_element_type=jnp.float32)

    b = jnp.reshape(batch_ref[...], (1, BLK))
    p = (lax.broadcasted_iota(jnp.int32, (G, BLK), 0) == b).astype(jnp.float32)
    yacc[...] = yacc[...] + jnp.dot(p, y, preferred_element_type=jnp.float32)
    cacc[...] = cacc[...] + jnp.sum(p, axis=1, keepdims=True)

    @pl.when(i == NGRID - 1)
    def _():
        pooled = yacc[...] / jnp.maximum(cacc[...], 1.0)
        logits = pooled + b4_ref[...]
        col = lax.broadcasted_iota(jnp.int32, (G, 128), 1)
        logits = jnp.where(col < C, logits, -1e30)
        mx = jnp.max(logits, axis=1, keepdims=True)
        e = jnp.exp(logits - mx)
        sm = e / jnp.sum(e, axis=1, keepdims=True)
        out_ref[...] = sm[:, :C]


def _row_spec(width):
    return pl.BlockSpec((BLK, width), lambda i: (i, 0))


def _whole(shape):
    nd = len(shape)
    return pl.BlockSpec(shape, lambda i: (0,) * nd)


def _mm_first(x, wd, wb, bias):
    k = x.shape[1]
    return pl.pallas_call(
        _mm_first_body,
        grid=(NGRID,),
        in_specs=[_row_spec(k), _whole((k, H)), _whole((k, H)),
                  _whole((1, H))],
        out_specs=[_row_spec(H), _row_spec(H)],
        out_shape=[jax.ShapeDtypeStruct((N, H), jnp.float32)] * 2,
    )(x, wd, wb, bias)


def _mm_bn(h, st, g, bt, wd, wb, bias):
    return pl.pallas_call(
        _mm_bn_body,
        grid=(NGRID,),
        in_specs=[_row_spec(H), _whole((8, H)), _whole((1, H)),
                  _whole((1, H)), _whole((H, H)), _whole((H, H)),
                  _whole((1, H))],
        out_specs=[_row_spec(H), _row_spec(H)],
        out_shape=[jax.ShapeDtypeStruct((N, H), jnp.float32)] * 2,
    )(h, st, g, bt, wd, wb, bias)


def _epilogue(a, m):
    return pl.pallas_call(
        _epi_body,
        grid=(NGRID,),
        in_specs=[_row_spec(H), _row_spec(H)],
        out_specs=[_row_spec(H), _whole((8, H))],
        out_shape=[jax.ShapeDtypeStruct((N, H), jnp.float32),
                   jax.ShapeDtypeStruct((8, H), jnp.float32)],
    )(a, m)


def _head(h, st, g, bt, batch3, w4p, b4p):
    return pl.pallas_call(
        _head_body,
        grid=(NGRID,),
        in_specs=[_row_spec(H), _whole((8, H)), _whole((1, H)),
                  _whole((1, H)),
                  pl.BlockSpec((1, 1, BLK), lambda i: (i, 0, 0)),
                  _whole((H, 128)), _whole((1, 128))],
        out_specs=pl.BlockSpec((G, C), lambda i: (0, 0)),
        out_shape=jax.ShapeDtypeStruct((G, C), jnp.float32),
        scratch_shapes=[pltpu.VMEM((G, 128), jnp.float32),
                        pltpu.VMEM((G, 128), jnp.float32)],
    )(h, st, g, bt, batch3, w4p, b4p)


def _layer_inputs(w, b, din):
    wd = w[:din] - w[din:]
    wb = w[din:]
    return wd, wb, b.reshape(1, H)


def _untranspose(mt):
    return jnp.transpose(mt, (1, 0, 2)).reshape(N, H)


def kernel(x, edge_index, batch, W1, b1, g1, bt1, W2, b2, g2, bt2,
           W3, b3, g3, bt3, W4, b4):
    src = edge_index[0]
    dst = edge_index[1]

    wd1, wb1, bb1 = _layer_inputs(W1, b1, D)
    wd2, wb2, bb2 = _layer_inputs(W2, b2, H)
    wd3, wb3, bb3 = _layer_inputs(W3, b3, H)
    w4p = jnp.zeros((H, 128), jnp.float32).at[:, :C].set(W4)
    b4p = jnp.zeros((1, 128), jnp.float32).at[:, :C].set(b4)
    batch3 = batch.reshape(NGRID, 1, BLK)

    bsrc, bdst, counts = _sc_bucket(src, dst)

    a1, bmat1 = _mm_first(x, wd1, wb1, bb1)
    m1 = _untranspose(
        _sc_edge_max(bmat1.reshape(N * NTILE, L), bsrc, bdst, counts))
    h1, st1 = _epilogue(a1, m1)

    a2, bmat2 = _mm_bn(h1, st1, g1.reshape(1, H), bt1.reshape(1, H),
                       wd2, wb2, bb2)
    m2 = _untranspose(
        _sc_edge_max(bmat2.reshape(N * NTILE, L), bsrc, bdst, counts))
    h2, st2 = _epilogue(a2, m2)

    a3, bmat3 = _mm_bn(h2, st2, g2.reshape(1, H), bt2.reshape(1, H),
                       wd3, wb3, bb3)
    m3 = _untranspose(
        _sc_edge_max(bmat3.reshape(N * NTILE, L), bsrc, bdst, counts))
    h3, st3 = _epilogue(a3, m3)

    return _head(h3, st3, g3.reshape(1, H), bt3.reshape(1, H),
                 batch3, w4p, b4p)


# speedup vs baseline: 2.2281x; 1.0779x over previous
import functools

import jax
import jax.numpy as jnp
from jax import lax
from jax.experimental import pallas as pl
from jax.experimental.pallas import tpu as pltpu
from jax.experimental.pallas import tpu_sc as plsc

N = 10000
E = 160000
D = 256
H = 512
G = 64
C = 10

NTILE = 32
L = 16
NH = N // 2
CH = 512
TE = E // NTILE
TEP = TE + 16
CAP = 5120
NREG = NTILE * 2
TABN = 544
TRASH = 528
BLK = 1000
NGRID = N // BLK
EPS = 1e-5
_SC_PARAMS = pltpu.CompilerParams(use_tc_tiling_on_sc=False,
                                  needs_layout_passes=False)


def _mesh():
    return plsc.VectorSubcoreMesh(core_axis_name="c", subcore_axis_name="s")


def _wid():
    return lax.axis_index("s") * 2 + lax.axis_index("c")


def _sc_bucket(src, dst):
    @functools.partial(
        pl.kernel,
        mesh=_mesh(),
        compiler_params=_SC_PARAMS,
        out_type=[
            jax.ShapeDtypeStruct((NTILE, 2, CAP), jnp.int32),
            jax.ShapeDtypeStruct((NTILE, 2, CAP), jnp.int32),
            jax.ShapeDtypeStruct((NTILE, L), jnp.int32),
        ],
        scratch_types=[
            pltpu.VMEM((TEP,), jnp.int32),
            pltpu.VMEM((TEP,), jnp.int32),
            pltpu.VMEM((CAP + L,), jnp.int32),
            pltpu.VMEM((CAP + L,), jnp.int32),
            pltpu.VMEM((CAP + L,), jnp.int32),
            pltpu.VMEM((CAP + L,), jnp.int32),
            pltpu.VMEM((L,), jnp.int32),
        ],
    )
    def k(src_hbm, dst_hbm, bsrc_hbm, bdst_hbm, cnt_hbm,
          sin, din, s0, d0, s1, d1, stage):
        wid = _wid()
        pltpu.sync_copy(src_hbm.at[pl.ds(wid * TE, TE)], sin.at[pl.ds(0, TE)])
        pltpu.sync_copy(dst_hbm.at[pl.ds(wid * TE, TE)], din.at[pl.ds(0, TE)])

        zeros = jnp.zeros((L,), jnp.int32)
        dumdst = jnp.full((L,), N, jnp.int32)

        def prefill(i, _):
            s0[pl.ds(i * L, L)] = zeros
            s1[pl.ds(i * L, L)] = zeros
            d0[pl.ds(i * L, L)] = dumdst
            d1[pl.ds(i * L, L)] = dumdst
            return 0

        lax.fori_loop(0, (CAP + L) // L, prefill, 0)

        sin[pl.ds(TE, L)] = zeros
        din[pl.ds(TE, L)] = dumdst

        iota = lax.iota(jnp.int32, L)
        ngrp = (TE + L - 1) // L

        def compact(j, carry):
            c0, c1 = carry
            sv = sin[pl.ds(j * L, L)]
            dv = din[pl.ds(j * L, L)]
            m1 = dv >= NH
            k1 = m1.astype(jnp.int32)
            k0 = 1 - k1
            p0 = plsc.cumsum(k0)
            p1 = plsc.cumsum(k1)
            idx0 = jnp.where(m1, CAP, c0 + p0 - 1)
            idx1 = jnp.where(m1, c1 + p1 - 1, CAP)
            plsc.store_scatter(s0, [idx0], sv)
            plsc.store_scatter(d0, [idx0], dv)
            plsc.store_scatter(s1, [idx1], sv)
            plsc.store_scatter(d1, [idx1], dv)
            n1 = jnp.sum(k1)
            return c0 + (L - n1), c1 + n1

        c0, c1 = lax.fori_loop(0, ngrp, compact, (jnp.int32(0), jnp.int32(0)))

        pltpu.sync_copy(s0.at[pl.ds(0, CAP)], bsrc_hbm.at[wid, 0])
        pltpu.sync_copy(d0.at[pl.ds(0, CAP)], bdst_hbm.at[wid, 0])
        pltpu.sync_copy(s1.at[pl.ds(0, CAP)], bsrc_hbm.at[wid, 1])
        pltpu.sync_copy(d1.at[pl.ds(0, CAP)], bdst_hbm.at[wid, 1])
        stage[...] = (jnp.where(iota == 0, c0, 0)
                      + jnp.where(iota == 1, c1, 0))
        pltpu.sync_copy(stage, cnt_hbm.at[wid])

    return k(src, dst)


def _sc_edge_max(btab, bsrc, bdst, counts):
    @functools.partial(
        pl.kernel,
        mesh=_mesh(),
        compiler_params=_SC_PARAMS,
        out_type=jax.ShapeDtypeStruct((NTILE, N, L), jnp.float32),
        scratch_types=[
            pltpu.VMEM((NTILE, L), jnp.int32),
            pltpu.VMEM((TABN,), jnp.int32),
            pltpu.VMEM((TABN,), jnp.int32),
            pltpu.VMEM((CH,), jnp.int32),
            pltpu.VMEM((CH,), jnp.int32),
            pltpu.VMEM((CH,), jnp.int32),
            pltpu.VMEM((CH,), jnp.int32),
            pltpu.VMEM((CH,), jnp.int32),
            pltpu.VMEM((CH,), jnp.int32),
            pltpu.VMEM((CH, L), jnp.float32),
            pltpu.VMEM((CH, L), jnp.float32),
            pltpu.VMEM((NH + 8, L), jnp.float32),
            pltpu.SemaphoreType.DMA,
            pltpu.SemaphoreType.DMA,
            pltpu.SemaphoreType.DMA,
            pltpu.SemaphoreType.DMA,
            pltpu.SemaphoreType.DMA,
            pltpu.SemaphoreType.DMA,
        ],
    )
    def k(btab_hbm, bsrc_hbm, bdst_hbm, cnt_hbm, mt_hbm,
          cnt_v, tab0, tab1, srcA, srcB, dstA, dstB, idxA, idxB,
          rowA, rowB, acc_v,
          sA, sB, dA, dB, gA, gB):
        wid = _wid()
        iota = lax.iota(jnp.int32, L)
        pltpu.sync_copy(cnt_hbm, cnt_v)

        srcs = (srcA, srcB)
        dsts = (dstA, dstB)
        idxs = (idxA, idxB)
        rows = (rowA, rowB)
        ssems = (sA, sB)
        dsems = (dA, dB)
        gsems = (gA, gB)
        tabs = (tab0, tab1)

        def build(p):
            def region(t2, nc):
                cnt = cnt_v[t2][p]
                ncr = (cnt + (CH - 1)) // CH
                base = (t2 * 2 + p) * CAP
                tidx = jnp.where(iota < ncr, nc + iota, TRASH)
                plsc.store_scatter(tabs[p], [tidx], base + iota * CH)
                return nc + ncr

            return lax.fori_loop(0, NTILE, region, jnp.int32(0))

        ncs = (build(0), build(1))

        def tab_at(p, c):
            cb = (c // 8) * 8
            v = tabs[p][pl.ds(cb, L)]
            raw = jnp.sum(jnp.where(iota == c - cb, v, 0))
            return (raw // 16) * 16

        def start_ids(p, c, b):
            off = tab_at(p, c)
            cs = pltpu.async_copy(
                bsrc_hbm.at[pl.ds(off, CH)], srcs[b], ssems[b])
            cd = pltpu.async_copy(
                bdst_hbm.at[pl.ds(off, CH)], dsts[b], dsems[b])
            return cs, cd

        def wait_ids(b):
            pltpu.make_async_copy(
                bsrc_hbm.at[pl.ds(0, CH)], srcs[b], ssems[b]).wait()
            pltpu.make_async_copy(
                bdst_hbm.at[pl.ds(0, CH)], dsts[b], dsems[b]).wait()

        def start_gather(b):
            def idx_body(j, _):
                s16 = srcs[b][pl.ds(j * L, L)]
                idxs[b][pl.ds(j * L, L)] = s16 * NTILE + wid
                return 0

            lax.fori_loop(0, CH // L, idx_body, 0)
            pltpu.async_copy(btab_hbm.at[idxs[b]], rows[b], gsems[b])

        def wait_gather(b):
            pltpu.make_async_copy(
                btab_hbm.at[idxs[b]], rows[b], gsems[b]).wait()

        for p in range(2):
            base = p * NH
            nc = ncs[p]

            neg = jnp.full((L,), -jnp.inf, jnp.float32)

            def init_body(i, _):
                acc_v[i] = neg
                return 0

            lax.fori_loop(0, NH, init_body, 0)

            @pl.when(nc > 0)
            def _():
                start_ids(p, 0, 0)
                wait_ids(0)
                start_gather(0)

            @pl.when(nc > 1)
            def _():
                start_ids(p, 1, 1)

            def rmw(c, b):
                def grp(j, _):
                    dv = dsts[b][pl.ds(j * L, L)]
                    dl = dv - base
                    dcv = jnp.where((dl >= 0) & (dl < NH), dl, NH)
                    for i in range(L):
                        dc = dcv[i]
                        r = rows[b][j * L + i]
                        acc_v[dc] = jnp.maximum(acc_v[dc], r)
                    return 0

                lax.fori_loop(0, CH // L, grp, 0)

            def body(c, cur, nxt):
                @pl.when(c < nc)
                def _():
                    @pl.when(c + 1 < nc)
                    def _():
                        wait_ids(nxt)
                        start_gather(nxt)

                    wait_gather(cur)
                    rmw(c, cur)

                    @pl.when(c + 2 < nc)
                    def _():
                        start_ids(p, c + 2, cur)

            def pair(kk, _):
                body(2 * kk, 0, 1)
                body(2 * kk + 1, 1, 0)
                return 0

            lax.fori_loop(0, (nc + 1) // 2, pair, 0)
            pltpu.sync_copy(acc_v.at[pl.ds(0, NH)], mt_hbm.at[wid, pl.ds(base, NH)])

    return k(btab, bsrc.reshape(-1), bdst.reshape(-1), counts)


def _mm_first_body(x_ref, wd_ref, wb_ref, bias_ref, a_ref, b_ref):
    z = x_ref[...]
    a_ref[...] = (
        jnp.dot(z, wd_ref[...], preferred_element_type=jnp.float32)
        + bias_ref[...]
    )
    b_ref[...] = jnp.dot(z, wb_ref[...], preferred_element_type=jnp.float32)


def _mm_bn_body(x_ref, st_ref, g_ref, bt_ref, wd_ref, wb_ref, bias_ref,
                a_ref, b_ref):
    m = st_ref[0:1, :] / N
    v = st_ref[1:2, :] / N - m * m
    sc = g_ref[...] * lax.rsqrt(v + EPS)
    sh = bt_ref[...] - m * sc
    z = jnp.maximum(x_ref[...] * sc + sh, 0.0)
    a_ref[...] = (
        jnp.dot(z, wd_ref[...], preferred_element_type=jnp.float32)
        + bias_ref[...]
    )
    b_ref[...] = jnp.dot(z, wb_ref[...], preferred_element_type=jnp.float32)


def _epi_body(a_ref, m_ref, h_ref, st_ref):
    h = jnp.maximum(a_ref[...] + m_ref[...], 0.0)
    h_ref[...] = h

    @pl.when(pl.program_id(0) == 0)
    def _():
        st_ref[...] = jnp.zeros_like(st_ref)

    st_ref[0:1, :] = st_ref[0:1, :] + jnp.sum(h, axis=0, keepdims=True)
    st_ref[1:2, :] = st_ref[1:2, :] + jnp.sum(h * h, axis=0, keepdims=True)


def _head_body(h_ref, st_ref, g_ref, bt_ref, batch_ref, w4_ref, b4_ref,
               out_ref, yacc, cacc):
    i = pl.program_id(0)

    @pl.when(i == 0)
    def _():
        yacc[...] = jnp.zeros_like(yacc)
        cacc[...] = jnp.zeros_like(cacc)

    m = st_ref[0:1, :] / N
    v = st_ref[1:2, :] / N - m * m
    sc = g_ref[...] * lax.rsqrt(v + EPS)
    sh = bt_ref[...] - m * sc
    z = h_ref[...] * sc + sh
    y = jnp.dot(z, w4_ref[...], preferred_element_type=jnp.float32)

    b = jnp.reshape(batch_ref[...], (1, BLK))
    p = (lax.broadcasted_iota(jnp.int32, (G, BLK), 0) == b).astype(jnp.float32)
    yacc[...] = yacc[...] + jnp.dot(p, y, preferred_element_type=jnp.float32)
    cacc[...] = cacc[...] + jnp.sum(p, axis=1, keepdims=True)

    @pl.when(i == NGRID - 1)
    def _():
        pooled = yacc[...] / jnp.maximum(cacc[...], 1.0)
        logits = pooled + b4_ref[...]
        col = lax.broadcasted_iota(jnp.int32, (G, 128), 1)
        logits = jnp.where(col < C, logits, -1e30)
        mx = jnp.max(logits, axis=1, keepdims=True)
        e = jnp.exp(logits - mx)
        sm = e / jnp.sum(e, axis=1, keepdims=True)
        out_ref[...] = sm[:, :C]


def _row_spec(width):
    return pl.BlockSpec((BLK, width), lambda i: (i, 0))


def _whole(shape):
    nd = len(shape)
    return pl.BlockSpec(shape, lambda i: (0,) * nd)


def _mm_first(x, wd, wb, bias):
    k = x.shape[1]
    return pl.pallas_call(
        _mm_first_body,
        grid=(NGRID,),
        in_specs=[_row_spec(k), _whole((k, H)), _whole((k, H)),
                  _whole((1, H))],
        out_specs=[_row_spec(H), _row_spec(H)],
        out_shape=[jax.ShapeDtypeStruct((N, H), jnp.float32)] * 2,
    )(x, wd, wb, bias)


def _mm_bn(h, st, g, bt, wd, wb, bias):
    return pl.pallas_call(
        _mm_bn_body,
        grid=(NGRID,),
        in_specs=[_row_spec(H), _whole((8, H)), _whole((1, H)),
                  _whole((1, H)), _whole((H, H)), _whole((H, H)),
                  _whole((1, H))],
        out_specs=[_row_spec(H), _row_spec(H)],
        out_shape=[jax.ShapeDtypeStruct((N, H), jnp.float32)] * 2,
    )(h, st, g, bt, wd, wb, bias)


def _epilogue(a, m):
    return pl.pallas_call(
        _epi_body,
        grid=(NGRID,),
        in_specs=[_row_spec(H), _row_spec(H)],
        out_specs=[_row_spec(H), _whole((8, H))],
        out_shape=[jax.ShapeDtypeStruct((N, H), jnp.float32),
                   jax.ShapeDtypeStruct((8, H), jnp.float32)],
    )(a, m)


def _head(h, st, g, bt, batch3, w4p, b4p):
    return pl.pallas_call(
        _head_body,
        grid=(NGRID,),
        in_specs=[_row_spec(H), _whole((8, H)), _whole((1, H)),
                  _whole((1, H)),
                  pl.BlockSpec((1, 1, BLK), lambda i: (i, 0, 0)),
                  _whole((H, 128)), _whole((1, 128))],
        out_specs=pl.BlockSpec((G, C), lambda i: (0, 0)),
        out_shape=jax.ShapeDtypeStruct((G, C), jnp.float32),
        scratch_shapes=[pltpu.VMEM((G, 128), jnp.float32),
                        pltpu.VMEM((G, 128), jnp.float32)],
    )(h, st, g, bt, batch3, w4p, b4p)


def _layer_inputs(w, b, din):
    wd = w[:din] - w[din:]
    wb = w[din:]
    return wd, wb, b.reshape(1, H)


def _untranspose(mt):
    return jnp.transpose(mt, (1, 0, 2)).reshape(N, H)


def kernel(x, edge_index, batch, W1, b1, g1, bt1, W2, b2, g2, bt2,
           W3, b3, g3, bt3, W4, b4):
    src = edge_index[0]
    dst = edge_index[1]

    wd1, wb1, bb1 = _layer_inputs(W1, b1, D)
    wd2, wb2, bb2 = _layer_inputs(W2, b2, H)
    wd3, wb3, bb3 = _layer_inputs(W3, b3, H)
    w4p = jnp.zeros((H, 128), jnp.float32).at[:, :C].set(W4)
    b4p = jnp.zeros((1, 128), jnp.float32).at[:, :C].set(b4)
    batch3 = batch.reshape(NGRID, 1, BLK)

    bsrc, bdst, counts = _sc_bucket(src, dst)

    a1, bmat1 = _mm_first(x, wd1, wb1, bb1)
    m1 = _untranspose(
        _sc_edge_max(bmat1.reshape(N * NTILE, L), bsrc, bdst, counts))
    h1, st1 = _epilogue(a1, m1)

    a2, bmat2 = _mm_bn(h1, st1, g1.reshape(1, H), bt1.reshape(1, H),
                       wd2, wb2, bb2)
    m2 = _untranspose(
        _sc_edge_max(bmat2.reshape(N * NTILE, L), bsrc, bdst, counts))
    h2, st2 = _epilogue(a2, m2)

    a3, bmat3 = _mm_bn(h2, st2, g2.reshape(1, H), bt2.reshape(1, H),
                       wd3, wb3, bb3)
    m3 = _untranspose(
        _sc_edge_max(bmat3.reshape(N * NTILE, L), bsrc, bdst, counts))
    h3, st3 = _epilogue(a3, m3)

    return _head(h3, st3, g3.reshape(1, H), bt3.reshape(1, H),
                 batch3, w4p, b4p)


# speedup vs baseline: 2.4533x; 1.1011x over previous
import functools

import jax
import jax.numpy as jnp
from jax import lax
from jax.experimental import pallas as pl
from jax.experimental.pallas import tpu as pltpu
from jax.experimental.pallas import tpu_sc as plsc

N = 10000
E = 160000
D = 256
H = 512
G = 64
C = 10

NTILE = 32
L = 16
NH = N // 2
CH = 512
TE = E // NTILE
TEP = TE + 16
CAP = 5120
NREG = NTILE * 2
TABN = 544
TRASH = 528
BLK = 1000
NGRID = N // BLK
EPS = 1e-5
_SC_PARAMS = pltpu.CompilerParams(use_tc_tiling_on_sc=False,
                                  needs_layout_passes=False)


def _mesh():
    return plsc.VectorSubcoreMesh(core_axis_name="c", subcore_axis_name="s")


def _wid():
    return lax.axis_index("s") * 2 + lax.axis_index("c")


def _sc_bucket(src, dst):
    @functools.partial(
        pl.kernel,
        mesh=_mesh(),
        compiler_params=_SC_PARAMS,
        out_type=[
            jax.ShapeDtypeStruct((NTILE, 2, CAP), jnp.int32),
            jax.ShapeDtypeStruct((NTILE, 2, CAP), jnp.int32),
            jax.ShapeDtypeStruct((NTILE, L), jnp.int32),
        ],
        scratch_types=[
            pltpu.VMEM((TEP,), jnp.int32),
            pltpu.VMEM((TEP,), jnp.int32),
            pltpu.VMEM((CAP + L,), jnp.int32),
            pltpu.VMEM((CAP + L,), jnp.int32),
            pltpu.VMEM((CAP + L,), jnp.int32),
            pltpu.VMEM((CAP + L,), jnp.int32),
            pltpu.VMEM((L,), jnp.int32),
        ],
    )
    def k(src_hbm, dst_hbm, bsrc_hbm, bdst_hbm, cnt_hbm,
          sin, din, s0, d0, s1, d1, stage):
        wid = _wid()
        pltpu.sync_copy(src_hbm.at[pl.ds(wid * TE, TE)], sin.at[pl.ds(0, TE)])
        pltpu.sync_copy(dst_hbm.at[pl.ds(wid * TE, TE)], din.at[pl.ds(0, TE)])

        zeros = jnp.zeros((L,), jnp.int32)
        dumdst = jnp.full((L,), N, jnp.int32)

        def prefill(i, _):
            s0[pl.ds(i * L, L)] = zeros
            s1[pl.ds(i * L, L)] = zeros
            d0[pl.ds(i * L, L)] = dumdst
            d1[pl.ds(i * L, L)] = dumdst
            return 0

        lax.fori_loop(0, (CAP + L) // L, prefill, 0)

        sin[pl.ds(TE, L)] = zeros
        din[pl.ds(TE, L)] = dumdst

        iota = lax.iota(jnp.int32, L)
        ngrp = (TE + L - 1) // L

        def compact(j, carry):
            c0, c1 = carry
            sv = sin[pl.ds(j * L, L)]
            dv = din[pl.ds(j * L, L)]
            m1 = dv >= NH
            k1 = m1.astype(jnp.int32)
            k0 = 1 - k1
            p0 = plsc.cumsum(k0)
            p1 = plsc.cumsum(k1)
            idx0 = jnp.where(m1, CAP, c0 + p0 - 1)
            idx1 = jnp.where(m1, c1 + p1 - 1, CAP)
            plsc.store_scatter(s0, [idx0], sv)
            plsc.store_scatter(d0, [idx0], dv)
            plsc.store_scatter(s1, [idx1], sv)
            plsc.store_scatter(d1, [idx1], dv)
            n1 = jnp.sum(k1)
            return c0 + (L - n1), c1 + n1

        c0, c1 = lax.fori_loop(0, ngrp, compact, (jnp.int32(0), jnp.int32(0)))

        pltpu.sync_copy(s0.at[pl.ds(0, CAP)], bsrc_hbm.at[wid, 0])
        pltpu.sync_copy(d0.at[pl.ds(0, CAP)], bdst_hbm.at[wid, 0])
        pltpu.sync_copy(s1.at[pl.ds(0, CAP)], bsrc_hbm.at[wid, 1])
        pltpu.sync_copy(d1.at[pl.ds(0, CAP)], bdst_hbm.at[wid, 1])
        stage[...] = (jnp.where(iota == 0, c0, 0)
                      + jnp.where(iota == 1, c1, 0))
        pltpu.sync_copy(stage, cnt_hbm.at[wid])

    return k(src, dst)


def _sc_edge_max(btab, bsrc, bdst, counts):
    @functools.partial(
        pl.kernel,
        mesh=_mesh(),
        compiler_params=_SC_PARAMS,
        out_type=jax.ShapeDtypeStruct((N, H), jnp.float32),
        scratch_types=[
            pltpu.VMEM((NTILE, L), jnp.int32),
            pltpu.VMEM((TABN,), jnp.int32),
            pltpu.VMEM((TABN,), jnp.int32),
            pltpu.VMEM((CH,), jnp.int32),
            pltpu.VMEM((CH,), jnp.int32),
            pltpu.VMEM((CH,), jnp.int32),
            pltpu.VMEM((CH,), jnp.int32),
            pltpu.VMEM((CH,), jnp.int32),
            pltpu.VMEM((CH,), jnp.int32),
            pltpu.VMEM((CH, L), jnp.float32),
            pltpu.VMEM((CH, L), jnp.float32),
            pltpu.VMEM((NH + 8, L), jnp.float32),
            pltpu.SemaphoreType.DMA,
            pltpu.SemaphoreType.DMA,
            pltpu.SemaphoreType.DMA,
            pltpu.SemaphoreType.DMA,
            pltpu.SemaphoreType.DMA,
            pltpu.SemaphoreType.DMA,
        ],
    )
    def k(btab_hbm, bsrc_hbm, bdst_hbm, cnt_hbm, mt_hbm,
          cnt_v, tab0, tab1, srcA, srcB, dstA, dstB, idxA, idxB,
          rowA, rowB, acc_v,
          sA, sB, dA, dB, gA, gB):
        wid = _wid()
        iota = lax.iota(jnp.int32, L)
        pltpu.sync_copy(cnt_hbm, cnt_v)

        srcs = (srcA, srcB)
        dsts = (dstA, dstB)
        idxs = (idxA, idxB)
        rows = (rowA, rowB)
        ssems = (sA, sB)
        dsems = (dA, dB)
        gsems = (gA, gB)
        tabs = (tab0, tab1)

        def build(p):
            def region(t2, nc):
                cnt = cnt_v[t2][p]
                ncr = (cnt + (CH - 1)) // CH
                base = (t2 * 2 + p) * CAP
                tidx = jnp.where(iota < ncr, nc + iota, TRASH)
                plsc.store_scatter(tabs[p], [tidx], base + iota * CH)
                return nc + ncr

            return lax.fori_loop(0, NTILE, region, jnp.int32(0))

        ncs = (build(0), build(1))

        def tab_at(p, c):
            cb = (c // 8) * 8
            v = tabs[p][pl.ds(cb, L)]
            raw = jnp.sum(jnp.where(iota == c - cb, v, 0))
            return (raw // 16) * 16

        def start_ids(p, c, b):
            off = tab_at(p, c)
            cs = pltpu.async_copy(
                bsrc_hbm.at[pl.ds(off, CH)], srcs[b], ssems[b])
            cd = pltpu.async_copy(
                bdst_hbm.at[pl.ds(off, CH)], dsts[b], dsems[b])
            return cs, cd

        def wait_ids(b):
            pltpu.make_async_copy(
                bsrc_hbm.at[pl.ds(0, CH)], srcs[b], ssems[b]).wait()
            pltpu.make_async_copy(
                bdst_hbm.at[pl.ds(0, CH)], dsts[b], dsems[b]).wait()

        def start_gather(b):
            def idx_body(j, _):
                s16 = srcs[b][pl.ds(j * L, L)]
                idxs[b][pl.ds(j * L, L)] = s16 * NTILE + wid
                return 0

            lax.fori_loop(0, CH // L, idx_body, 0)
            pltpu.async_copy(btab_hbm.at[idxs[b]], rows[b], gsems[b])

        def wait_gather(b):
            pltpu.make_async_copy(
                btab_hbm.at[idxs[b]], rows[b], gsems[b]).wait()

        for p in range(2):
            base = p * NH
            nc = ncs[p]

            neg = jnp.full((L,), -jnp.inf, jnp.float32)

            def init_body(i, _):
                acc_v[i] = neg
                return 0

            lax.fori_loop(0, NH, init_body, 0)

            @pl.when(nc > 0)
            def _():
                start_ids(p, 0, 0)
                wait_ids(0)
                start_gather(0)

            @pl.when(nc > 1)
            def _():
                start_ids(p, 1, 1)

            def rmw(c, b):
                def grp(j, _):
                    dv = dsts[b][pl.ds(j * L, L)]
                    dl = dv - base
                    dcv = jnp.where((dl >= 0) & (dl < NH), dl, NH)
                    for i in range(L):
                        dc = dcv[i]
                        r = rows[b][j * L + i]
                        acc_v[dc] = jnp.maximum(acc_v[dc], r)
                    return 0

                lax.fori_loop(0, CH // L, grp, 0)

            def body(c, cur, nxt):
                @pl.when(c < nc)
                def _():
                    @pl.when(c + 1 < nc)
                    def _():
                        wait_ids(nxt)
                        start_gather(nxt)

                    wait_gather(cur)
                    rmw(c, cur)

                    @pl.when(c + 2 < nc)
                    def _():
                        start_ids(p, c + 2, cur)

            def pair(kk, _):
                body(2 * kk, 0, 1)
                body(2 * kk + 1, 1, 0)
                return 0

            lax.fori_loop(0, (nc + 1) // 2, pair, 0)
            pltpu.sync_copy(
                acc_v.at[pl.ds(0, NH)],
                mt_hbm.at[pl.ds(base, NH), pl.ds(wid * L, L)])

    return k(btab, bsrc.reshape(-1), bdst.reshape(-1), counts)


def _mm_first_body(x_ref, wd_ref, wb_ref, bias_ref, a_ref, b_ref):
    z = x_ref[...]
    a_ref[...] = (
        jnp.dot(z, wd_ref[...], preferred_element_type=jnp.float32)
        + bias_ref[...]
    )
    b_ref[...] = jnp.dot(z, wb_ref[...], preferred_element_type=jnp.float32)


def _mm_bn_body(x_ref, st_ref, g_ref, bt_ref, wd_ref, wb_ref, bias_ref,
                a_ref, b_ref):
    m = st_ref[0:1, :] / N
    v = st_ref[1:2, :] / N - m * m
    sc = g_ref[...] * lax.rsqrt(v + EPS)
    sh = bt_ref[...] - m * sc
    z = jnp.maximum(x_ref[...] * sc + sh, 0.0)
    a_ref[...] = (
        jnp.dot(z, wd_ref[...], preferred_element_type=jnp.float32)
        + bias_ref[...]
    )
    b_ref[...] = jnp.dot(z, wb_ref[...], preferred_element_type=jnp.float32)


def _epi_body(a_ref, m_ref, h_ref, st_ref):
    h = jnp.maximum(a_ref[...] + m_ref[...], 0.0)
    h_ref[...] = h

    @pl.when(pl.program_id(0) == 0)
    def _():
        st_ref[...] = jnp.zeros_like(st_ref)

    st_ref[0:1, :] = st_ref[0:1, :] + jnp.sum(h, axis=0, keepdims=True)
    st_ref[1:2, :] = st_ref[1:2, :] + jnp.sum(h * h, axis=0, keepdims=True)


def _head_body(h_ref, st_ref, g_ref, bt_ref, batch_ref, w4_ref, b4_ref,
               out_ref, yacc, cacc):
    i = pl.program_id(0)

    @pl.when(i == 0)
    def _():
        yacc[...] = jnp.zeros_like(yacc)
        cacc[...] = jnp.zeros_like(cacc)

    m = st_ref[0:1, :] / N
    v = st_ref[1:2, :] / N - m * m
    sc = g_ref[...] * lax.rsqrt(v + EPS)
    sh = bt_ref[...] - m * sc
    z = h_ref[...] * sc + sh
    y = jnp.dot(z, w4_ref[...], preferred_element_type=jnp.float32)

    b = jnp.reshape(batch_ref[...], (1, BLK))
    p = (lax.broadcasted_iota(jnp.int32, (G, BLK), 0) == b).astype(jnp.float32)
    yacc[...] = yacc[...] + jnp.dot(p, y, preferred_element_type=jnp.float32)
    cacc[...] = cacc[...] + jnp.sum(p, axis=1, keepdims=True)

    @pl.when(i == NGRID - 1)
    def _():
        pooled = yacc[...] / jnp.maximum(cacc[...], 1.0)
        logits = pooled + b4_ref[...]
        col = lax.broadcasted_iota(jnp.int32, (G, 128), 1)
        logits = jnp.where(col < C, logits, -1e30)
        mx = jnp.max(logits, axis=1, keepdims=True)
        e = jnp.exp(logits - mx)
        sm = e / jnp.sum(e, axis=1, keepdims=True)
        out_ref[...] = sm[:, :C]


def _row_spec(width):
    return pl.BlockSpec((BLK, width), lambda i: (i, 0))


def _whole(shape):
    nd = len(shape)
    return pl.BlockSpec(shape, lambda i: (0,) * nd)


def _mm_first(x, wd, wb, bias):
    k = x.shape[1]
    return pl.pallas_call(
        _mm_first_body,
        grid=(NGRID,),
        in_specs=[_row_spec(k), _whole((k, H)), _whole((k, H)),
                  _whole((1, H))],
        out_specs=[_row_spec(H), _row_spec(H)],
        out_shape=[jax.ShapeDtypeStruct((N, H), jnp.float32)] * 2,
    )(x, wd, wb, bias)


def _mm_bn(h, st, g, bt, wd, wb, bias):
    return pl.pallas_call(
        _mm_bn_body,
        grid=(NGRID,),
        in_specs=[_row_spec(H), _whole((8, H)), _whole((1, H)),
                  _whole((1, H)), _whole((H, H)), _whole((H, H)),
                  _whole((1, H))],
        out_specs=[_row_spec(H), _row_spec(H)],
        out_shape=[jax.ShapeDtypeStruct((N, H), jnp.float32)] * 2,
    )(h, st, g, bt, wd, wb, bias)


def _epilogue(a, m):
    return pl.pallas_call(
        _epi_body,
        grid=(NGRID,),
        in_specs=[_row_spec(H), _row_spec(H)],
        out_specs=[_row_spec(H), _whole((8, H))],
        out_shape=[jax.ShapeDtypeStruct((N, H), jnp.float32),
                   jax.ShapeDtypeStruct((8, H), jnp.float32)],
    )(a, m)


def _head(h, st, g, bt, batch3, w4p, b4p):
    return pl.pallas_call(
        _head_body,
        grid=(NGRID,),
        in_specs=[_row_spec(H), _whole((8, H)), _whole((1, H)),
                  _whole((1, H)),
                  pl.BlockSpec((1, 1, BLK), lambda i: (i, 0, 0)),
                  _whole((H, 128)), _whole((1, 128))],
        out_specs=pl.BlockSpec((G, C), lambda i: (0, 0)),
        out_shape=jax.ShapeDtypeStruct((G, C), jnp.float32),
        scratch_shapes=[pltpu.VMEM((G, 128), jnp.float32),
                        pltpu.VMEM((G, 128), jnp.float32)],
    )(h, st, g, bt, batch3, w4p, b4p)


def _layer_inputs(w, b, din):
    wd = w[:din] - w[din:]
    wb = w[din:]
    return wd, wb, b.reshape(1, H)


def kernel(x, edge_index, batch, W1, b1, g1, bt1, W2, b2, g2, bt2,
           W3, b3, g3, bt3, W4, b4):
    src = edge_index[0]
    dst = edge_index[1]

    wd1, wb1, bb1 = _layer_inputs(W1, b1, D)
    wd2, wb2, bb2 = _layer_inputs(W2, b2, H)
    wd3, wb3, bb3 = _layer_inputs(W3, b3, H)
    w4p = jnp.zeros((H, 128), jnp.float32).at[:, :C].set(W4)
    b4p = jnp.zeros((1, 128), jnp.float32).at[:, :C].set(b4)
    batch3 = batch.reshape(NGRID, 1, BLK)

    bsrc, bdst, counts = _sc_bucket(src, dst)

    a1, bmat1 = _mm_first(x, wd1, wb1, bb1)
    m1 = _sc_edge_max(bmat1.reshape(N * NTILE, L), bsrc, bdst, counts)
    h1, st1 = _epilogue(a1, m1)

    a2, bmat2 = _mm_bn(h1, st1, g1.reshape(1, H), bt1.reshape(1, H),
                       wd2, wb2, bb2)
    m2 = _sc_edge_max(bmat2.reshape(N * NTILE, L), bsrc, bdst, counts)
    h2, st2 = _epilogue(a2, m2)

    a3, bmat3 = _mm_bn(h2, st2, g2.reshape(1, H), bt2.reshape(1, H),
                       wd3, wb3, bb3)
    m3 = _sc_edge_max(bmat3.reshape(N * NTILE, L), bsrc, bdst, counts)
    h3, st3 = _epilogue(a3, m3)

    return _head(h3, st3, g3.reshape(1, H), bt3.reshape(1, H),
                 batch3, w4p, b4p)


# speedup vs baseline: 2.5322x; 1.0322x over previous
import functools

import jax
import jax.numpy as jnp
from jax import lax
from jax.experimental import pallas as pl
from jax.experimental.pallas import tpu as pltpu
from jax.experimental.pallas import tpu_sc as plsc

N = 10000
E = 160000
D = 256
H = 512
G = 64
C = 10

NTILE = 32
L = 16
NH = N // 2
CH = 512
TE = E // NTILE
TEP = TE + 16
CAP = 5120
NREG = NTILE * 2
TABN = 544
TRASH = 528
BLK = 1000
NGRID = N // BLK
EPS = 1e-5
_SC_PARAMS = pltpu.CompilerParams(use_tc_tiling_on_sc=False,
                                  needs_layout_passes=False)


def _mesh():
    return plsc.VectorSubcoreMesh(core_axis_name="c", subcore_axis_name="s")


def _wid():
    return lax.axis_index("s") * 2 + lax.axis_index("c")


def _sc_bucket(src, dst):
    @functools.partial(
        pl.kernel,
        mesh=_mesh(),
        compiler_params=_SC_PARAMS,
        out_type=[
            jax.ShapeDtypeStruct((NTILE, 2, CAP), jnp.int32),
            jax.ShapeDtypeStruct((NTILE, 2, CAP), jnp.int32),
            jax.ShapeDtypeStruct((NTILE, L), jnp.int32),
        ],
        scratch_types=[
            pltpu.VMEM((TEP,), jnp.int32),
            pltpu.VMEM((TEP,), jnp.int32),
            pltpu.VMEM((CAP + L,), jnp.int32),
            pltpu.VMEM((CAP + L,), jnp.int32),
            pltpu.VMEM((CAP + L,), jnp.int32),
            pltpu.VMEM((CAP + L,), jnp.int32),
            pltpu.VMEM((L,), jnp.int32),
        ],
    )
    def k(src_hbm, dst_hbm, bsrc_hbm, bdst_hbm, cnt_hbm,
          sin, din, s0, d0, s1, d1, stage):
        wid = _wid()
        pltpu.sync_copy(src_hbm.at[pl.ds(wid * TE, TE)], sin.at[pl.ds(0, TE)])
        pltpu.sync_copy(dst_hbm.at[pl.ds(wid * TE, TE)], din.at[pl.ds(0, TE)])

        zeros = jnp.zeros((L,), jnp.int32)
        dumdst = jnp.full((L,), N, jnp.int32)

        def prefill(i, _):
            s0[pl.ds(i * L, L)] = zeros
            s1[pl.ds(i * L, L)] = zeros
            d0[pl.ds(i * L, L)] = dumdst
            d1[pl.ds(i * L, L)] = dumdst
            return 0

        lax.fori_loop(0, (CAP + L) // L, prefill, 0)

        sin[pl.ds(TE, L)] = zeros
        din[pl.ds(TE, L)] = dumdst

        iota = lax.iota(jnp.int32, L)
        ngrp = (TE + L - 1) // L

        def compact(j, carry):
            c0, c1 = carry
            sv = sin[pl.ds(j * L, L)]
            dv = din[pl.ds(j * L, L)]
            m1 = dv >= NH
            k1 = m1.astype(jnp.int32)
            k0 = 1 - k1
            p0 = plsc.cumsum(k0)
            p1 = plsc.cumsum(k1)
            idx0 = jnp.where(m1, CAP, c0 + p0 - 1)
            idx1 = jnp.where(m1, c1 + p1 - 1, CAP)
            plsc.store_scatter(s0, [idx0], sv)
            plsc.store_scatter(d0, [idx0], dv)
            plsc.store_scatter(s1, [idx1], sv)
            plsc.store_scatter(d1, [idx1], dv)
            n1 = jnp.sum(k1)
            return c0 + (L - n1), c1 + n1

        c0, c1 = lax.fori_loop(0, ngrp, compact, (jnp.int32(0), jnp.int32(0)))

        pltpu.sync_copy(s0.at[pl.ds(0, CAP)], bsrc_hbm.at[wid, 0])
        pltpu.sync_copy(d0.at[pl.ds(0, CAP)], bdst_hbm.at[wid, 0])
        pltpu.sync_copy(s1.at[pl.ds(0, CAP)], bsrc_hbm.at[wid, 1])
        pltpu.sync_copy(d1.at[pl.ds(0, CAP)], bdst_hbm.at[wid, 1])
        stage[...] = (jnp.where(iota == 0, c0, 0)
                      + jnp.where(iota == 1, c1, 0))
        pltpu.sync_copy(stage, cnt_hbm.at[wid])

    return k(src, dst)


def _sc_edge_max(btab, bsrc, bdst, counts):
    @functools.partial(
        pl.kernel,
        mesh=_mesh(),
        compiler_params=_SC_PARAMS,
        out_type=jax.ShapeDtypeStruct((N, H), jnp.float32),
        scratch_types=[
            pltpu.VMEM((NTILE, L), jnp.int32),
            pltpu.VMEM((TABN,), jnp.int32),
            pltpu.VMEM((TABN,), jnp.int32),
            pltpu.VMEM((CH,), jnp.int32),
            pltpu.VMEM((CH,), jnp.int32),
            pltpu.VMEM((CH,), jnp.int32),
            pltpu.VMEM((CH,), jnp.int32),
            pltpu.VMEM((CH,), jnp.int32),
            pltpu.VMEM((CH,), jnp.int32),
            pltpu.VMEM((CH, L), jnp.float32),
            pltpu.VMEM((CH, L), jnp.float32),
            pltpu.VMEM((NH + 8, L), jnp.float32),
            pltpu.SemaphoreType.DMA,
            pltpu.SemaphoreType.DMA,
            pltpu.SemaphoreType.DMA,
            pltpu.SemaphoreType.DMA,
            pltpu.SemaphoreType.DMA,
            pltpu.SemaphoreType.DMA,
        ],
    )
    def k(btab_hbm, bsrc_hbm, bdst_hbm, cnt_hbm, mt_hbm,
          cnt_v, tab0, tab1, srcA, srcB, dstA, dstB, idxA, idxB,
          rowA, rowB, acc_v,
          sA, sB, dA, dB, gA, gB):
        wid = _wid()
        iota = lax.iota(jnp.int32, L)
        pltpu.sync_copy(cnt_hbm, cnt_v)

        srcs = (srcA, srcB)
        dsts = (dstA, dstB)
        idxs = (idxA, idxB)
        rows = (rowA, rowB)
        ssems = (sA, sB)
        dsems = (dA, dB)
        gsems = (gA, gB)
        tabs = (tab0, tab1)

        def build(p):
            def region(t2, nc):
                cnt = cnt_v[t2][p]
                ncr = (cnt + (CH - 1)) // CH
                base = (t2 * 2 + p) * CAP
                tidx = jnp.where(iota < ncr, nc + iota, TRASH)
                plsc.store_scatter(tabs[p], [tidx], base + iota * CH)
                return nc + ncr

            return lax.fori_loop(0, NTILE, region, jnp.int32(0))

        ncs = (build(0), build(1))

        def tab_at(p, c):
            cb = (c // 8) * 8
            v = tabs[p][pl.ds(cb, L)]
            raw = jnp.sum(jnp.where(iota == c - cb, v, 0))
            return (raw // 16) * 16

        def start_ids(p, c, b):
            off = tab_at(p, c)
            cs = pltpu.async_copy(
                bsrc_hbm.at[pl.ds(off, CH)], srcs[b], ssems[b])
            cd = pltpu.async_copy(
                bdst_hbm.at[pl.ds(off, CH)], dsts[b], dsems[b])
            return cs, cd

        def wait_ids(b):
            pltpu.make_async_copy(
                bsrc_hbm.at[pl.ds(0, CH)], srcs[b], ssems[b]).wait()
            pltpu.make_async_copy(
                bdst_hbm.at[pl.ds(0, CH)], dsts[b], dsems[b]).wait()

        def start_gather(b):
            def idx_body(j, _):
                s16 = srcs[b][pl.ds(j * L, L)]
                idxs[b][pl.ds(j * L, L)] = s16 * NTILE + wid
                return 0

            lax.fori_loop(0, CH // L, idx_body, 0)
            pltpu.async_copy(btab_hbm.at[idxs[b]], rows[b], gsems[b])

        def wait_gather(b):
            pltpu.make_async_copy(
                btab_hbm.at[idxs[b]], rows[b], gsems[b]).wait()

        for p in range(2):
            base = p * NH
            nc = ncs[p]

            neg = jnp.full((L,), -jnp.inf, jnp.float32)
            acc_v[NH] = neg
            acc_v[NH + 1] = neg

            def init_body(i, _):
                for u in range(4):
                    acc_v[i * 4 + u] = neg
                return 0

            lax.fori_loop(0, NH // 4, init_body, 0)

            @pl.when(nc > 0)
            def _():
                start_ids(p, 0, 0)
                wait_ids(0)
                start_gather(0)

            @pl.when(nc > 1)
            def _():
                start_ids(p, 1, 1)

            def rmw(c, b):
                def grp(j, _):
                    dv = dsts[b][pl.ds(j * L, L)]
                    dl = dv - base
                    dcv = jnp.where((dl >= 0) & (dl < NH), dl, NH)
                    for i in range(L):
                        dc = dcv[i]
                        r = rows[b][j * L + i]
                        acc_v[dc] = jnp.maximum(acc_v[dc], r)
                    return 0

                lax.fori_loop(0, CH // L, grp, 0, unroll=2)

            def body(c, cur, nxt):
                @pl.when(c < nc)
                def _():
                    @pl.when(c + 1 < nc)
                    def _():
                        wait_ids(nxt)
                        start_gather(nxt)

                    wait_gather(cur)
                    rmw(c, cur)

                    @pl.when(c + 2 < nc)
                    def _():
                        start_ids(p, c + 2, cur)

            def pair(kk, _):
                body(2 * kk, 0, 1)
                body(2 * kk + 1, 1, 0)
                return 0

            lax.fori_loop(0, (nc + 1) // 2, pair, 0)
            pltpu.sync_copy(
                acc_v.at[pl.ds(0, NH)],
                mt_hbm.at[pl.ds(base, NH), pl.ds(wid * L, L)])

    return k(btab, bsrc.reshape(-1), bdst.reshape(-1), counts)


def _mm_first_body(x_ref, wd_ref, wb_ref, bias_ref, a_ref, b_ref):
    z = x_ref[...]
    a_ref[...] = (
        jnp.dot(z, wd_ref[...], preferred_element_type=jnp.float32)
        + bias_ref[...]
    )
    b_ref[...] = jnp.dot(z, wb_ref[...], preferred_element_type=jnp.float32)


def _mm_bn_body(x_ref, st_ref, g_ref, bt_ref, wd_ref, wb_ref, bias_ref,
                a_ref, b_ref):
    m = st_ref[0:1, :] / N
    v = st_ref[1:2, :] / N - m * m
    sc = g_ref[...] * lax.rsqrt(v + EPS)
    sh = bt_ref[...] - m * sc
    z = jnp.maximum(x_ref[...] * sc + sh, 0.0)
    a_ref[...] = (
        jnp.dot(z, wd_ref[...], preferred_element_type=jnp.float32)
        + bias_ref[...]
    )
    b_ref[...] = jnp.dot(z, wb_ref[...], preferred_element_type=jnp.float32)


def _epi_body(a_ref, m_ref, h_ref, st_ref):
    h = jnp.maximum(a_ref[...] + m_ref[...], 0.0)
    h_ref[...] = h

    @pl.when(pl.program_id(0) == 0)
    def _():
        st_ref[...] = jnp.zeros_like(st_ref)

    st_ref[0:1, :] = st_ref[0:1, :] + jnp.sum(h, axis=0, keepdims=True)
    st_ref[1:2, :] = st_ref[1:2, :] + jnp.sum(h * h, axis=0, keepdims=True)


def _head_body(h_ref, st_ref, g_ref, bt_ref, batch_ref, w4_ref, b4_ref,
               out_ref, yacc, cacc):
    i = pl.program_id(0)

    @pl.when(i == 0)
    def _():
        yacc[...] = jnp.zeros_like(yacc)
        cacc[...] = jnp.zeros_like(cacc)

    m = st_ref[0:1, :] / N
    v = st_ref[1:2, :] / N - m * m
    sc = g_ref[...] * lax.rsqrt(v + EPS)
    sh = bt_ref[...] - m * sc
    z = h_ref[...] * sc + sh
    y = jnp.dot(z, w4_ref[...], preferred_element_type=jnp.float32)

    b = jnp.reshape(batch_ref[...], (1, BLK))
    p = (lax.broadcasted_iota(jnp.int32, (G, BLK), 0) == b).astype(jnp.float32)
    yacc[...] = yacc[...] + jnp.dot(p, y, preferred_element_type=jnp.float32)
    cacc[...] = cacc[...] + jnp.sum(p, axis=1, keepdims=True)

    @pl.when(i == NGRID - 1)
    def _():
        pooled = yacc[...] / jnp.maximum(cacc[...], 1.0)
        logits = pooled + b4_ref[...]
        col = lax.broadcasted_iota(jnp.int32, (G, 128), 1)
        logits = jnp.where(col < C, logits, -1e30)
        mx = jnp.max(logits, axis=1, keepdims=True)
        e = jnp.exp(logits - mx)
        sm = e / jnp.sum(e, axis=1, keepdims=True)
        out_ref[...] = sm[:, :C]


def _row_spec(width):
    return pl.BlockSpec((BLK, width), lambda i: (i, 0))


def _whole(shape):
    nd = len(shape)
    return pl.BlockSpec(shape, lambda i: (0,) * nd)


def _mm_first(x, wd, wb, bias):
    k = x.shape[1]
    return pl.pallas_call(
        _mm_first_body,
        grid=(NGRID,),
        in_specs=[_row_spec(k), _whole((k, H)), _whole((k, H)),
                  _whole((1, H))],
        out_specs=[_row_spec(H), _row_spec(H)],
        out_shape=[jax.ShapeDtypeStruct((N, H), jnp.float32)] * 2,
    )(x, wd, wb, bias)


def _mm_bn(h, st, g, bt, wd, wb, bias):
    return pl.pallas_call(
        _mm_bn_body,
        grid=(NGRID,),
        in_specs=[_row_spec(H), _whole((8, H)), _whole((1, H)),
                  _whole((1, H)), _whole((H, H)), _whole((H, H)),
                  _whole((1, H))],
        out_specs=[_row_spec(H), _row_spec(H)],
        out_shape=[jax.ShapeDtypeStruct((N, H), jnp.float32)] * 2,
    )(h, st, g, bt, wd, wb, bias)


def _epilogue(a, m):
    return pl.pallas_call(
        _epi_body,
        grid=(NGRID,),
        in_specs=[_row_spec(H), _row_spec(H)],
        out_specs=[_row_spec(H), _whole((8, H))],
        out_shape=[jax.ShapeDtypeStruct((N, H), jnp.float32),
                   jax.ShapeDtypeStruct((8, H), jnp.float32)],
    )(a, m)


def _head(h, st, g, bt, batch3, w4p, b4p):
    return pl.pallas_call(
        _head_body,
        grid=(NGRID,),
        in_specs=[_row_spec(H), _whole((8, H)), _whole((1, H)),
                  _whole((1, H)),
                  pl.BlockSpec((1, 1, BLK), lambda i: (i, 0, 0)),
                  _whole((H, 128)), _whole((1, 128))],
        out_specs=pl.BlockSpec((G, C), lambda i: (0, 0)),
        out_shape=jax.ShapeDtypeStruct((G, C), jnp.float32),
        scratch_shapes=[pltpu.VMEM((G, 128), jnp.float32),
                        pltpu.VMEM((G, 128), jnp.float32)],
    )(h, st, g, bt, batch3, w4p, b4p)


def _layer_inputs(w, b, din):
    wd = w[:din] - w[din:]
    wb = w[din:]
    return wd, wb, b.reshape(1, H)


def kernel(x, edge_index, batch, W1, b1, g1, bt1, W2, b2, g2, bt2,
           W3, b3, g3, bt3, W4, b4):
    src = edge_index[0]
    dst = edge_index[1]

    wd1, wb1, bb1 = _layer_inputs(W1, b1, D)
    wd2, wb2, bb2 = _layer_inputs(W2, b2, H)
    wd3, wb3, bb3 = _layer_inputs(W3, b3, H)
    w4p = jnp.zeros((H, 128), jnp.float32).at[:, :C].set(W4)
    b4p = jnp.zeros((1, 128), jnp.float32).at[:, :C].set(b4)
    batch3 = batch.reshape(NGRID, 1, BLK)

    bsrc, bdst, counts = _sc_bucket(src, dst)

    a1, bmat1 = _mm_first(x, wd1, wb1, bb1)
    m1 = _sc_edge_max(bmat1.reshape(N * NTILE, L), bsrc, bdst, counts)
    h1, st1 = _epilogue(a1, m1)

    a2, bmat2 = _mm_bn(h1, st1, g1.reshape(1, H), bt1.reshape(1, H),
                       wd2, wb2, bb2)
    m2 = _sc_edge_max(bmat2.reshape(N * NTILE, L), bsrc, bdst, counts)
    h2, st2 = _epilogue(a2, m2)

    a3, bmat3 = _mm_bn(h2, st2, g2.reshape(1, H), bt2.reshape(1, H),
                       wd3, wb3, bb3)
    m3 = _sc_edge_max(bmat3.reshape(N * NTILE, L), bsrc, bdst, counts)
    h3, st3 = _epilogue(a3, m3)

    return _head(h3, st3, g3.reshape(1, H), bt3.reshape(1, H),
                 batch3, w4p, b4p)
